# baseline (speedup 1.0000x reference)
import functools

import jax
import jax.numpy as jnp
from jax import lax
from jax.experimental import pallas as pl
from jax.experimental.pallas import tpu as pltpu
from jax.experimental.pallas import tpu_sc as plsc

_D = 64
_DP = 128
_B = 4096 * 200
_NC, _NS = 2, 16
_NW = _NC * _NS
_BPW = _B // _NW
_C = 256
_NCHUNK = _BPW // _C
_NBUF = 2
_INFLIGHT = 1


def _sc_gather(idx_flat, weight_pad):
    mesh = plsc.VectorSubcoreMesh(core_axis_name="c", subcore_axis_name="s")

    @functools.partial(
        pl.kernel,
        out_type=jax.ShapeDtypeStruct((_B, _DP), jnp.float32),
        mesh=mesh,
        scratch_types=[
            pltpu.VMEM((_BPW,), jnp.int32),
            pltpu.VMEM((_NBUF, _C, _DP), jnp.float32),
            [pltpu.SemaphoreType.DMA] * _NBUF,
            [pltpu.SemaphoreType.DMA] * _NBUF,
        ],
        compiler_params=pltpu.CompilerParams(use_tc_tiling_on_sc=True),
    )
    def k(weight_hbm, idx_hbm, out_hbm, idx_v, rows_v, gsem, wsem):
        wid = lax.axis_index("s") * _NC + lax.axis_index("c")
        base = wid * _BPW
        pltpu.sync_copy(idx_hbm.at[pl.ds(base, _BPW)], idx_v)

        def g_src(g):
            return weight_hbm.at[idx_v.at[pl.ds(g * _C, _C)]]

        def w_src(b):
            return rows_v.at[b]

        def w_dst(g):
            return out_hbm.at[pl.ds(base + g * _C, _C)]

        def gstart(g, b):
            pltpu.async_copy(g_src(g), rows_v.at[b], gsem[b])

        def gwait(g, b):
            pltpu.make_async_copy(g_src(g), rows_v.at[b], gsem[b]).wait()

        def wstart(g, b):
            pltpu.async_copy(w_src(b), w_dst(g), wsem[b])

        def wwait(g, b):
            pltpu.make_async_copy(w_src(b), w_dst(g), wsem[b]).wait()

        for i in range(_INFLIGHT):
            gstart(i, i)

        @pl.loop(0, _NCHUNK, step=_NBUF)
        def _outer(g0):
            for b in range(_NBUF):
                g = g0 + b
                gwait(g, b)
                wstart(g, b)
                nxt = g + _INFLIGHT
                b2 = (b + _INFLIGHT) % _NBUF

                @pl.when(nxt < _NCHUNK)
                def _():
                    prev = nxt - _NBUF

                    @pl.when(prev >= 0)
                    def _():
                        wwait(prev, b2)

                    gstart(nxt, b2)

        for b in range(_NBUF):
            wwait(_NCHUNK - _NBUF + b, b)

    return k(weight_pad, idx_flat)


_V = 1000000
_R = 512
_NBLK = -(-_V // _R)


def _tc_transpose_pad(weight):
    wt = weight.T

    def body(in_ref, out_ref):
        x = in_ref[...]
        eye = (
            lax.broadcasted_iota(jnp.int32, (_D, _DP), 0)
            == lax.broadcasted_iota(jnp.int32, (_D, _DP), 1)
        ).astype(jnp.float32)
        out_ref[...] = lax.dot_general(
            x,
            eye,
            dimension_numbers=(((0,), (0,)), ((), ())),
            preferred_element_type=jnp.float32,
            precision=lax.Precision.HIGHEST,
        )

    return pl.pallas_call(
        body,
        grid=(_NBLK,),
        in_specs=[pl.BlockSpec((_D, _R), lambda i: (0, i))],
        out_specs=pl.BlockSpec((_R, _DP), lambda i: (i, 0)),
        out_shape=jax.ShapeDtypeStruct((_V, _DP), jnp.float32),
    )(wt)


def kernel(input_ids, weight):
    idx = input_ids.reshape(-1).astype(jnp.int32)
    wp = _tc_transpose_pad(weight)
    out = _sc_gather(idx, wp)
    return out[:, :_D].reshape(*input_ids.shape, _D)


# speedup vs baseline: 2.5062x; 2.5062x over previous
import functools

import jax
import jax.numpy as jnp
from jax import lax
from jax.experimental import pallas as pl
from jax.experimental.pallas import tpu as pltpu
from jax.experimental.pallas import tpu_sc as plsc

_D = 64
_DP = 128
_B = 4096 * 200
_NC, _NS = 2, 16
_NW = _NC * _NS
_BPW = _B // _NW
_C = 256
_NCHUNK = _BPW // _C
_NBUF = 2
_INFLIGHT = 1


def _sc_gather(idx_flat, weight_pad):
    mesh = plsc.VectorSubcoreMesh(core_axis_name="c", subcore_axis_name="s")

    @functools.partial(
        pl.kernel,
        out_type=jax.ShapeDtypeStruct((_B, _DP), jnp.float32),
        mesh=mesh,
        scratch_types=[
            pltpu.VMEM((_BPW,), jnp.int32),
            pltpu.VMEM((_NBUF, _C, _DP), jnp.float32),
            [pltpu.SemaphoreType.DMA] * _NBUF,
            [pltpu.SemaphoreType.DMA] * _NBUF,
        ],
        compiler_params=pltpu.CompilerParams(use_tc_tiling_on_sc=True),
    )
    def k(weight_hbm, idx_hbm, out_hbm, idx_v, rows_v, gsem, wsem):
        wid = lax.axis_index("s") * _NC + lax.axis_index("c")
        base = wid * _BPW
        pltpu.sync_copy(idx_hbm.at[pl.ds(base, _BPW)], idx_v)

        def g_src(g):
            return weight_hbm.at[idx_v.at[pl.ds(g * _C, _C)]]

        def w_src(b):
            return rows_v.at[b]

        def w_dst(g):
            return out_hbm.at[pl.ds(base + g * _C, _C)]

        def gstart(g, b):
            pltpu.async_copy(g_src(g), rows_v.at[b], gsem[b])

        def gwait(g, b):
            pltpu.make_async_copy(g_src(g), rows_v.at[b], gsem[b]).wait()

        def wstart(g, b):
            pltpu.async_copy(w_src(b), w_dst(g), wsem[b])

        def wwait(g, b):
            pltpu.make_async_copy(w_src(b), w_dst(g), wsem[b]).wait()

        for i in range(_INFLIGHT):
            gstart(i, i)

        @pl.loop(0, _NCHUNK, step=_NBUF)
        def _outer(g0):
            for b in range(_NBUF):
                g = g0 + b
                gwait(g, b)
                wstart(g, b)
                nxt = g + _INFLIGHT
                b2 = (b + _INFLIGHT) % _NBUF

                @pl.when(nxt < _NCHUNK)
                def _():
                    prev = nxt - _NBUF

                    @pl.when(prev >= 0)
                    def _():
                        wwait(prev, b2)

                    gstart(nxt, b2)

        for b in range(_NBUF):
            wwait(_NCHUNK - _NBUF + b, b)

    return k(weight_pad, idx_flat)


_V = 1000000
_R = 8192
_NBLK = -(-_V // _R)


def _tc_transpose_pad(weight):
    wt = weight.T

    def body(in_ref, out_ref):
        x = in_ref[...]
        xt = x.T
        out_ref[...] = jnp.concatenate(
            [xt, jnp.zeros((_R, _DP - _D), jnp.float32)], axis=1
        )

    return pl.pallas_call(
        body,
        grid=(_NBLK,),
        in_specs=[pl.BlockSpec((_D, _R), lambda i: (0, i))],
        out_specs=pl.BlockSpec((_R, _DP), lambda i: (i, 0)),
        out_shape=jax.ShapeDtypeStruct((_V, _DP), jnp.float32),
    )(wt)


def kernel(input_ids, weight):
    idx = input_ids.reshape(-1).astype(jnp.int32)
    wp = _tc_transpose_pad(weight)
    out = _sc_gather(idx, wp)
    return out[:, :_D].reshape(*input_ids.shape, _D)


# speedup vs baseline: 2.5756x; 1.0277x over previous
import functools

import jax
import jax.numpy as jnp
from jax import lax
from jax.experimental import pallas as pl
from jax.experimental.pallas import tpu as pltpu
from jax.experimental.pallas import tpu_sc as plsc

_D = 64
_DP = 128
_B = 4096 * 200
_NC, _NS = 2, 16
_NW = _NC * _NS
_BPW = _B // _NW
_C = 200
_NCHUNK = _BPW // _C
_NBUF = 4
_INFLIGHT = 2


def _sc_gather(idx_flat, weight_pad):
    mesh = plsc.VectorSubcoreMesh(core_axis_name="c", subcore_axis_name="s")

    @functools.partial(
        pl.kernel,
        out_type=jax.ShapeDtypeStruct((_B, _DP), jnp.float32),
        mesh=mesh,
        scratch_types=[
            pltpu.VMEM((_BPW,), jnp.int32),
            pltpu.VMEM((_NBUF, _C, _DP), jnp.float32),
            [pltpu.SemaphoreType.DMA] * _NBUF,
            [pltpu.SemaphoreType.DMA] * _NBUF,
        ],
        compiler_params=pltpu.CompilerParams(use_tc_tiling_on_sc=True),
    )
    def k(weight_hbm, idx_hbm, out_hbm, idx_v, rows_v, gsem, wsem):
        wid = lax.axis_index("s") * _NC + lax.axis_index("c")
        base = wid * _BPW
        pltpu.sync_copy(idx_hbm.at[pl.ds(base, _BPW)], idx_v)

        def g_src(g):
            return weight_hbm.at[idx_v.at[pl.ds(g * _C, _C)]]

        def w_src(b):
            return rows_v.at[b]

        def w_dst(g):
            return out_hbm.at[pl.ds(base + g * _C, _C)]

        def gstart(g, b):
            pltpu.async_copy(g_src(g), rows_v.at[b], gsem[b])

        def gwait(g, b):
            pltpu.make_async_copy(g_src(g), rows_v.at[b], gsem[b]).wait()

        def wstart(g, b):
            pltpu.async_copy(w_src(b), w_dst(g), wsem[b])

        def wwait(g, b):
            pltpu.make_async_copy(w_src(b), w_dst(g), wsem[b]).wait()

        for i in range(_INFLIGHT):
            gstart(i, i)

        @pl.loop(0, _NCHUNK, step=_NBUF)
        def _outer(g0):
            for b in range(_NBUF):
                g = g0 + b
                gwait(g, b)
                wstart(g, b)
                nxt = g + _INFLIGHT
                b2 = (b + _INFLIGHT) % _NBUF

                @pl.when(nxt < _NCHUNK)
                def _():
                    prev = nxt - _NBUF

                    @pl.when(prev >= 0)
                    def _():
                        wwait(prev, b2)

                    gstart(nxt, b2)

        for b in range(_NBUF):
            wwait(_NCHUNK - _NBUF + b, b)

    return k(weight_pad, idx_flat)


_V = 1000000
_R = 16384
_NBLK = -(-_V // _R)


def _tc_transpose_pad(weight):
    wt = weight.T

    def body(in_ref, out_ref):
        x = in_ref[...]
        xt = x.T
        out_ref[...] = jnp.concatenate(
            [xt, jnp.zeros((_R, _DP - _D), jnp.float32)], axis=1
        )

    return pl.pallas_call(
        body,
        grid=(_NBLK,),
        in_specs=[pl.BlockSpec((_D, _R), lambda i: (0, i))],
        out_specs=pl.BlockSpec((_R, _DP), lambda i: (i, 0)),
        out_shape=jax.ShapeDtypeStruct((_V, _DP), jnp.float32),
    )(wt)


def kernel(input_ids, weight):
    idx = input_ids.reshape(-1).astype(jnp.int32)
    wp = _tc_transpose_pad(weight)
    out = _sc_gather(idx, wp)
    return out[:, :_D].reshape(*input_ids.shape, _D)
